# baseline (speedup 1.0000x reference)
import jax
import jax.numpy as jnp
from jax.experimental import pallas as pl
from jax.experimental.pallas import tpu as pltpu

HIDDEN = 64
MAX_DEPTH = 8
NUM_TYPES = 16
K = 32

BLK_TOK = 32768
M = BLK_TOK // 2


def _body(de_ref, do_ref, ce_ref, co_ref, dtab_ref, ttab_ref, w_ref, b_ref, out_ref):
    w = w_ref[...]
    pd = jax.lax.dot_general(dtab_ref[...], w[:, :HIDDEN],
                             (((1,), (1,)), ((), ())),
                             preferred_element_type=jnp.float32) + b_ref[...]
    pt = jax.lax.dot_general(ttab_ref[...], w[:, HIDDEN:],
                             (((1,), (1,)), ((), ())),
                             preferred_element_type=jnp.float32)
    z8 = jnp.zeros((K - MAX_DEPTH - NUM_TYPES, HIDDEN), jnp.float32)
    ptab = jnp.concatenate([pd, pt, z8], axis=0)
    hi = ptab.astype(jnp.bfloat16)
    lo = (ptab - hi.astype(jnp.float32)).astype(jnp.bfloat16)
    zk = jnp.zeros((K, HIDDEN), jnp.bfloat16)
    rhs_hi = jnp.concatenate(
        [jnp.concatenate([hi, zk], axis=1), jnp.concatenate([zk, hi], axis=1)], axis=0)
    rhs_lo = jnp.concatenate(
        [jnp.concatenate([lo, zk], axis=1), jnp.concatenate([zk, lo], axis=1)], axis=0)

    iota = jax.lax.broadcasted_iota(jnp.int32, (K, M), 0)
    de = jnp.broadcast_to(de_ref[0], (K, M))
    ce = jnp.broadcast_to(ce_ref[0] + MAX_DEPTH, (K, M))
    do = jnp.broadcast_to(do_ref[0], (K, M))
    co = jnp.broadcast_to(co_ref[0] + MAX_DEPTH, (K, M))
    th_e = jnp.where((iota == de) | (iota == ce), 1.0, 0.0)
    th_o = jnp.where((iota == do) | (iota == co), 1.0, 0.0)
    th = jnp.concatenate([th_e, th_o], axis=0).astype(jnp.bfloat16)

    out_ref[...] = (
        jax.lax.dot_general(th, rhs_hi, (((0,), (0,)), ((), ())),
                            preferred_element_type=jnp.float32)
        + jax.lax.dot_general(th, rhs_lo, (((0,), (0,)), ((), ())),
                              preferred_element_type=jnp.float32))


def kernel(depth_indices, node_type_indices, depth_table, type_table, W, b):
    B, L = depth_indices.shape
    n_tok = B * L
    grid = n_tok // BLK_TOK
    d2 = depth_indices.reshape(-1, 2)
    c2 = node_type_indices.reshape(-1, 2)
    de = d2[:, 0].reshape(grid, 1, M)
    do = d2[:, 1].reshape(grid, 1, M)
    ce = c2[:, 0].reshape(grid, 1, M)
    co = c2[:, 1].reshape(grid, 1, M)

    idx_spec = pl.BlockSpec((1, 1, M), lambda i: (i, 0, 0))
    out = pl.pallas_call(
        _body,
        grid=(grid,),
        in_specs=[
            idx_spec, idx_spec, idx_spec, idx_spec,
            pl.BlockSpec((MAX_DEPTH, HIDDEN), lambda i: (0, 0)),
            pl.BlockSpec((NUM_TYPES, HIDDEN), lambda i: (0, 0)),
            pl.BlockSpec((HIDDEN, 2 * HIDDEN), lambda i: (0, 0)),
            pl.BlockSpec((1, HIDDEN), lambda i: (0, 0)),
        ],
        out_specs=pl.BlockSpec((M, 2 * HIDDEN), lambda i: (i, 0)),
        out_shape=jax.ShapeDtypeStruct((n_tok // 2, 2 * HIDDEN), jnp.float32),
        compiler_params=pltpu.CompilerParams(
            dimension_semantics=("arbitrary",)),
    )(de, do, ce, co, depth_table, type_table, W, b.reshape(1, HIDDEN))
    return out.reshape(B, L, HIDDEN)


# speedup vs baseline: 3.2799x; 3.2799x over previous
import jax
import jax.numpy as jnp
from jax.experimental import pallas as pl
from jax.experimental.pallas import tpu as pltpu

HIDDEN = 64
MAX_DEPTH = 8
NUM_TYPES = 16
K = 32

BLK_TOK = 32768


def _body(didx_ref, tidx_ref, dtab_ref, ttab_ref, w_ref, b_ref, out_ref):
    w = w_ref[...]
    pd = jax.lax.dot_general(dtab_ref[...], w[:, :HIDDEN],
                             (((1,), (1,)), ((), ())),
                             preferred_element_type=jnp.float32) + b_ref[...]
    pt = jax.lax.dot_general(ttab_ref[...], w[:, HIDDEN:],
                             (((1,), (1,)), ((), ())),
                             preferred_element_type=jnp.float32)
    ptab = jnp.concatenate(
        [pd, pt, jnp.zeros((K - MAX_DEPTH - NUM_TYPES, HIDDEN), jnp.float32)], axis=0)

    d = jnp.broadcast_to(didx_ref[0], (K, BLK_TOK))
    c = jnp.broadcast_to(tidx_ref[0] + MAX_DEPTH, (K, BLK_TOK))
    iota = jax.lax.broadcasted_iota(jnp.int32, (K, BLK_TOK), 0)
    two_hot_t = jnp.where((iota == d) | (iota == c), 1.0, 0.0)
    out_ref[...] = jax.lax.dot_general(two_hot_t, ptab,
                                       (((0,), (0,)), ((), ())),
                                       preferred_element_type=jnp.float32)


def kernel(depth_indices, node_type_indices, depth_table, type_table, W, b):
    B, L = depth_indices.shape
    n_tok = B * L
    grid = n_tok // BLK_TOK
    di = depth_indices.reshape(grid, 1, BLK_TOK)
    ci = node_type_indices.reshape(grid, 1, BLK_TOK)

    out = pl.pallas_call(
        _body,
        grid=(grid,),
        in_specs=[
            pl.BlockSpec((1, 1, BLK_TOK), lambda i: (i, 0, 0)),
            pl.BlockSpec((1, 1, BLK_TOK), lambda i: (i, 0, 0)),
            pl.BlockSpec((MAX_DEPTH, HIDDEN), lambda i: (0, 0)),
            pl.BlockSpec((NUM_TYPES, HIDDEN), lambda i: (0, 0)),
            pl.BlockSpec((HIDDEN, 2 * HIDDEN), lambda i: (0, 0)),
            pl.BlockSpec((1, HIDDEN), lambda i: (0, 0)),
        ],
        out_specs=pl.BlockSpec((BLK_TOK, HIDDEN), lambda i: (i, 0)),
        out_shape=jax.ShapeDtypeStruct((n_tok, HIDDEN), jnp.float32),
        compiler_params=pltpu.CompilerParams(
            dimension_semantics=("parallel",)),
    )(di, ci, depth_table, type_table, W, b.reshape(1, HIDDEN))
    return out.reshape(B, L, HIDDEN)


# speedup vs baseline: 3.3743x; 1.0288x over previous
import jax
import jax.numpy as jnp
from jax.experimental import pallas as pl
from jax.experimental.pallas import tpu as pltpu

HIDDEN = 64
MAX_DEPTH = 8
NUM_TYPES = 16
K = 32

BLK_TOK = 32768


def _body(comb_ref, dtab_ref, ttab_ref, w_ref, b_ref, out_ref):
    w = w_ref[...]
    pd = jax.lax.dot_general(dtab_ref[...], w[:, :HIDDEN],
                             (((1,), (1,)), ((), ())),
                             preferred_element_type=jnp.float32) + b_ref[...]
    pt = jax.lax.dot_general(ttab_ref[...], w[:, HIDDEN:],
                             (((1,), (1,)), ((), ())),
                             preferred_element_type=jnp.float32)
    ptab = jnp.concatenate(
        [pd, pt, jnp.zeros((K - MAX_DEPTH - NUM_TYPES, HIDDEN), jnp.float32)], axis=0)

    row = comb_ref[0]
    d = jnp.broadcast_to(row & (MAX_DEPTH - 1), (K, BLK_TOK))
    c = jnp.broadcast_to((row >> 3) + MAX_DEPTH, (K, BLK_TOK))
    iota = jax.lax.broadcasted_iota(jnp.int32, (K, BLK_TOK), 0)
    two_hot_t = jnp.where((iota == d) | (iota == c), 1.0, 0.0)
    out_ref[...] = jax.lax.dot_general(two_hot_t, ptab,
                                       (((0,), (0,)), ((), ())),
                                       preferred_element_type=jnp.float32)


def kernel(depth_indices, node_type_indices, depth_table, type_table, W, b):
    B, L = depth_indices.shape
    n_tok = B * L
    grid = n_tok // BLK_TOK
    comb = ((node_type_indices << 3) | depth_indices).reshape(grid, 1, BLK_TOK)

    out = pl.pallas_call(
        _body,
        grid=(grid,),
        in_specs=[
            pl.BlockSpec((1, 1, BLK_TOK), lambda i: (i, 0, 0)),
            pl.BlockSpec((MAX_DEPTH, HIDDEN), lambda i: (0, 0)),
            pl.BlockSpec((NUM_TYPES, HIDDEN), lambda i: (0, 0)),
            pl.BlockSpec((HIDDEN, 2 * HIDDEN), lambda i: (0, 0)),
            pl.BlockSpec((1, HIDDEN), lambda i: (0, 0)),
        ],
        out_specs=pl.BlockSpec((BLK_TOK, HIDDEN), lambda i: (i, 0)),
        out_shape=jax.ShapeDtypeStruct((n_tok, HIDDEN), jnp.float32),
        compiler_params=pltpu.CompilerParams(
            dimension_semantics=("parallel",)),
    )(comb, depth_table, type_table, W, b.reshape(1, HIDDEN))
    return out.reshape(B, L, HIDDEN)


# speedup vs baseline: 3.3936x; 1.0057x over previous
import jax
import jax.numpy as jnp
from jax.experimental import pallas as pl
from jax.experimental.pallas import tpu as pltpu

HIDDEN = 64
MAX_DEPTH = 8
NUM_TYPES = 16
K = 32

BLK_TOK = 32768


def _body(comb_ref, dtab_ref, ttab_ref, w_ref, b_ref, out_ref):
    w = w_ref[...]
    pd = jax.lax.dot_general(dtab_ref[...], w[:, :HIDDEN],
                             (((1,), (1,)), ((), ())),
                             preferred_element_type=jnp.float32) + b_ref[...]
    pt = jax.lax.dot_general(ttab_ref[...], w[:, HIDDEN:],
                             (((1,), (1,)), ((), ())),
                             preferred_element_type=jnp.float32)
    ptab = jnp.concatenate(
        [pd, pt, jnp.zeros((K - MAX_DEPTH - NUM_TYPES, HIDDEN), jnp.float32)], axis=0)

    row = comb_ref[0].astype(jnp.int32)
    d = jnp.broadcast_to(row & (MAX_DEPTH - 1), (K, BLK_TOK))
    c = jnp.broadcast_to((row >> 3) + MAX_DEPTH, (K, BLK_TOK))
    iota = jax.lax.broadcasted_iota(jnp.int32, (K, BLK_TOK), 0)
    two_hot_t = jnp.where((iota == d) | (iota == c), 1.0, 0.0)
    out_ref[...] = jax.lax.dot_general(two_hot_t, ptab,
                                       (((0,), (0,)), ((), ())),
                                       preferred_element_type=jnp.float32)


def kernel(depth_indices, node_type_indices, depth_table, type_table, W, b):
    B, L = depth_indices.shape
    n_tok = B * L
    grid = n_tok // BLK_TOK
    comb = ((node_type_indices << 3) | depth_indices).astype(jnp.int8).reshape(
        grid, 1, BLK_TOK)

    out = pl.pallas_call(
        _body,
        grid=(grid,),
        in_specs=[
            pl.BlockSpec((1, 1, BLK_TOK), lambda i: (i, 0, 0)),
            pl.BlockSpec((MAX_DEPTH, HIDDEN), lambda i: (0, 0)),
            pl.BlockSpec((NUM_TYPES, HIDDEN), lambda i: (0, 0)),
            pl.BlockSpec((HIDDEN, 2 * HIDDEN), lambda i: (0, 0)),
            pl.BlockSpec((1, HIDDEN), lambda i: (0, 0)),
        ],
        out_specs=pl.BlockSpec((BLK_TOK, HIDDEN), lambda i: (i, 0)),
        out_shape=jax.ShapeDtypeStruct((n_tok, HIDDEN), jnp.float32),
        compiler_params=pltpu.CompilerParams(
            dimension_semantics=("parallel",)),
    )(comb, depth_table, type_table, W, b.reshape(1, HIDDEN))
    return out.reshape(B, L, HIDDEN)
